# baseline (speedup 1.0000x reference)
import jax
import jax.numpy as jnp
from jax import lax
from jax.experimental import pallas as pl
from jax.experimental.pallas import tpu as pltpu
from jax.experimental.pallas import tpu_sc as plsc

_BATCH = 16384
_DIM = 32
_NUM_WORKERS = 32
_B_PER_W = _BATCH // _NUM_WORKERS
_CHUNK = 128
_NCHUNK = _B_PER_W // _CHUNK
_PHYS_ROWS = 250000


def _mf_body(user_ids_hbm, item_ids_hbm, user_emb_hbm, item_emb_hbm,
             out_hbm, uid_v, iid_v, pu_idx, pi_idx, ubuf, ibuf, out_v,
             sem_u, sem_i):
    num_cores = 2
    wid = lax.axis_index("s") * num_cores + lax.axis_index("c")
    base = wid * _B_PER_W

    pltpu.sync_copy(user_ids_hbm.at[pl.ds(base, _B_PER_W)], uid_v)
    pltpu.sync_copy(item_ids_hbm.at[pl.ds(base, _B_PER_W)], iid_v)

    for k in range(_B_PER_W // 16):
        c, o = k // (_CHUNK // 16), (k % (_CHUNK // 16)) * 16
        pu_idx[c, pl.ds(o, 16)] = uid_v[pl.ds(k * 16, 16)] >> 2
        pi_idx[c, pl.ds(o, 16)] = iid_v[pl.ds(k * 16, 16)] >> 2

    lane = lax.iota(jnp.int32, 16)

    def chunk(c, _):
        cp_u = pltpu.async_copy(user_emb_hbm.at[pu_idx.at[c]], ubuf, sem_u)
        cp_i = pltpu.async_copy(item_emb_hbm.at[pi_idx.at[c]], ibuf, sem_i)
        cp_u.wait()
        cp_i.wait()

        def group(g, _):
            cbase = c * _CHUNK + g * 16
            row16 = g * 16 + lane
            uoff = (uid_v[pl.ds(cbase, 16)] & 3) << 5
            ioff = (iid_v[pl.ds(cbase, 16)] & 3) << 5
            acc = jnp.zeros((16,), jnp.float32)
            for j in range(_DIM):
                uu = plsc.load_gather(ubuf, [row16, uoff + j])
                vv = plsc.load_gather(ibuf, [row16, ioff + j])
                acc = acc + uu * vv
            out_v[pl.ds(cbase, 16)] = acc
            return _

        lax.fori_loop(0, _CHUNK // 16, group, None)
        return _

    lax.fori_loop(0, _NCHUNK, chunk, None)

    pltpu.sync_copy(out_v, out_hbm.at[pl.ds(base, _B_PER_W)])


@jax.jit
def _mf(user_ids, item_ids, user_emb, item_emb):
    mesh = plsc.VectorSubcoreMesh(core_axis_name="c", subcore_axis_name="s")
    ue = user_emb.reshape(_PHYS_ROWS, 128)
    ie = item_emb.reshape(_PHYS_ROWS, 128)
    return pl.kernel(
        _mf_body,
        out_type=jax.ShapeDtypeStruct((_BATCH,), jnp.float32),
        mesh=mesh,
        scratch_types=[
            pltpu.VMEM((_B_PER_W,), jnp.int32),
            pltpu.VMEM((_B_PER_W,), jnp.int32),
            pltpu.VMEM((_NCHUNK, _CHUNK), jnp.int32),
            pltpu.VMEM((_NCHUNK, _CHUNK), jnp.int32),
            pltpu.VMEM((_CHUNK, 128), jnp.float32),
            pltpu.VMEM((_CHUNK, 128), jnp.float32),
            pltpu.VMEM((_B_PER_W,), jnp.float32),
            pltpu.SemaphoreType.DMA,
            pltpu.SemaphoreType.DMA,
        ],
        compiler_params=pltpu.CompilerParams(needs_layout_passes=False),
    )(user_ids, item_ids, ue, ie)


def kernel(user_ids, item_ids, user_emb, item_emb):
    return _mf(user_ids, item_ids, user_emb, item_emb)


# speedup vs baseline: 3.6047x; 3.6047x over previous
import jax
import jax.numpy as jnp
from jax import lax
from jax.experimental import pallas as pl
from jax.experimental.pallas import tpu as pltpu
from jax.experimental.pallas import tpu_sc as plsc

_BATCH = 16384
_DIM = 32
_NUM_WORKERS = 32
_B_PER_W = _BATCH // _NUM_WORKERS
_CHUNK = 8
_NCHUNK = _B_PER_W // _CHUNK
_NROWS = 1000000
_PAD_ROWS = 1000064


def _mf_body(user_ids_hbm, item_ids_hbm, uemb_t_hbm, iemb_t_hbm,
             out_hbm, uid_v, iid_v, ubuf, ibuf, out_v,
             sem_u, sem_i):
    num_cores = 2
    wid = lax.axis_index("s") * num_cores + lax.axis_index("c")
    base = wid * _B_PER_W

    pltpu.sync_copy(user_ids_hbm.at[pl.ds(base, _B_PER_W)],
                    uid_v.at[pl.ds(0, _B_PER_W)])
    pltpu.sync_copy(item_ids_hbm.at[pl.ds(base, _B_PER_W)],
                    iid_v.at[pl.ds(0, _B_PER_W)])
    lane = lax.iota(jnp.int32, 16)
    row_j = (lane & 7) * _DIM

    def fire_chunk(c):
        uid = uid_v[pl.ds(c * _CHUNK, 16)]
        iid = iid_v[pl.ds(c * _CHUNK, 16)]
        ubv = (uid >> 7) << 7
        ibv = (iid >> 7) << 7
        for b in range(_CHUNK):
            ub = jnp.sum(jnp.where(lane == b, ubv, 0))
            ib = jnp.sum(jnp.where(lane == b, ibv, 0))
            pltpu.async_copy(
                uemb_t_hbm.at[:, pl.ds(pl.multiple_of(ub, 128), 128)],
                ubuf.at[pl.ds(b * _DIM, _DIM), :], sem_u)
            pltpu.async_copy(
                iemb_t_hbm.at[:, pl.ds(pl.multiple_of(ib, 128), 128)],
                ibuf.at[pl.ds(b * _DIM, _DIM), :], sem_i)

    def drain_chunk():
        for b in range(_CHUNK):
            pltpu.make_async_copy(
                uemb_t_hbm.at[:, pl.ds(0, 128)],
                ubuf.at[pl.ds(b * _DIM, _DIM), :], sem_u).wait()
            pltpu.make_async_copy(
                iemb_t_hbm.at[:, pl.ds(0, 128)],
                ibuf.at[pl.ds(b * _DIM, _DIM), :], sem_i).wait()

    def compute_chunk(c):
        uid = uid_v[pl.ds(c * _CHUNK, 16)]
        iid = iid_v[pl.ds(c * _CHUNK, 16)]
        ucol = uid & 127
        icol = iid & 127
        acc = jnp.zeros((16,), jnp.float32)
        for j in range(_DIM):
            uu = plsc.load_gather(ubuf, [row_j + j, ucol])
            vv = plsc.load_gather(ibuf, [row_j + j, icol])
            acc = acc + uu * vv
        return acc

    def super_chunk(sc, _):
        fire_chunk(sc * 2)
        drain_chunk()
        acc_even = compute_chunk(sc * 2)
        fire_chunk(sc * 2 + 1)
        drain_chunk()
        acc_odd = compute_chunk(sc * 2 + 1)
        shifted = lax.gather(
            acc_odd, (lane & 7)[:, None],
            dimension_numbers=lax.GatherDimensionNumbers(
                offset_dims=(), collapsed_slice_dims=(0,),
                start_index_map=(0,)),
            slice_sizes=(1,),
            mode=lax.GatherScatterMode.PROMISE_IN_BOUNDS)
        out_v[pl.ds(sc * 16, 16)] = jnp.where(lane < 8, acc_even, shifted)
        return _

    lax.fori_loop(0, _NCHUNK // 2, super_chunk, None)

    pltpu.sync_copy(out_v, out_hbm.at[pl.ds(base, _B_PER_W)])


@jax.jit
def _mf(user_ids, item_ids, user_emb, item_emb):
    mesh = plsc.VectorSubcoreMesh(core_axis_name="c", subcore_axis_name="s")
    return pl.kernel(
        _mf_body,
        out_type=jax.ShapeDtypeStruct((_BATCH,), jnp.float32),
        mesh=mesh,
        scratch_types=[
            pltpu.VMEM((_B_PER_W + 16,), jnp.int32),
            pltpu.VMEM((_B_PER_W + 16,), jnp.int32),
            pltpu.VMEM((_CHUNK * _DIM, 128), jnp.float32),
            pltpu.VMEM((_CHUNK * _DIM, 128), jnp.float32),
            pltpu.VMEM((_B_PER_W,), jnp.float32),
            pltpu.SemaphoreType.DMA,
            pltpu.SemaphoreType.DMA,
        ],
        compiler_params=pltpu.CompilerParams(
            needs_layout_passes=False, disable_bounds_checks=True),
    )(user_ids, item_ids, user_emb.T, item_emb.T)


def kernel(user_ids, item_ids, user_emb, item_emb):
    return _mf(user_ids, item_ids, user_emb, item_emb)


# speedup vs baseline: 3.9027x; 1.0827x over previous
import jax
import jax.numpy as jnp
from jax import lax
from jax.experimental import pallas as pl
from jax.experimental.pallas import tpu as pltpu
from jax.experimental.pallas import tpu_sc as plsc

_BATCH = 16384
_DIM = 32
_NUM_WORKERS = 32
_B_PER_W = _BATCH // _NUM_WORKERS
_CHUNK = 4
_NCHUNK = _B_PER_W // _CHUNK


def _mf_body(user_ids_hbm, item_ids_hbm, uemb_t_hbm, iemb_t_hbm,
             out_hbm, uid_v, iid_v, ubuf0, ibuf0, ubuf1, ibuf1, out_v,
             sem_u0, sem_i0, sem_u1, sem_i1):
    num_cores = 2
    wid = lax.axis_index("s") * num_cores + lax.axis_index("c")
    base = wid * _B_PER_W

    pltpu.sync_copy(user_ids_hbm.at[pl.ds(base, _B_PER_W)],
                    uid_v.at[pl.ds(0, _B_PER_W)])
    pltpu.sync_copy(item_ids_hbm.at[pl.ds(base, _B_PER_W)],
                    iid_v.at[pl.ds(0, _B_PER_W)])
    lane = lax.iota(jnp.int32, 16)
    row_j = (lane & (_CHUNK - 1)) * _DIM
    bufs = ((ubuf0, ibuf0, sem_u0, sem_i0), (ubuf1, ibuf1, sem_u1, sem_i1))

    def fire_chunk(c, slot):
        ubuf, ibuf, sem_u, sem_i = bufs[slot]
        uid = uid_v[pl.ds(c * _CHUNK, 16)]
        iid = iid_v[pl.ds(c * _CHUNK, 16)]
        ubv = (uid >> 7) << 7
        ibv = (iid >> 7) << 7
        for b in range(_CHUNK):
            ub = jnp.sum(jnp.where(lane == b, ubv, 0))
            ib = jnp.sum(jnp.where(lane == b, ibv, 0))
            pltpu.async_copy(
                uemb_t_hbm.at[:, pl.ds(pl.multiple_of(ub, 128), 128)],
                ubuf.at[pl.ds(b * _DIM, _DIM), :], sem_u)
            pltpu.async_copy(
                iemb_t_hbm.at[:, pl.ds(pl.multiple_of(ib, 128), 128)],
                ibuf.at[pl.ds(b * _DIM, _DIM), :], sem_i)

    def drain_compute_chunk(c, slot):
        ubuf, ibuf, sem_u, sem_i = bufs[slot]
        for b in range(_CHUNK):
            pltpu.make_async_copy(
                uemb_t_hbm.at[:, pl.ds(0, 128)],
                ubuf.at[pl.ds(b * _DIM, _DIM), :], sem_u).wait()
            pltpu.make_async_copy(
                iemb_t_hbm.at[:, pl.ds(0, 128)],
                ibuf.at[pl.ds(b * _DIM, _DIM), :], sem_i).wait()
        uid = uid_v[pl.ds(c * _CHUNK, 16)]
        iid = iid_v[pl.ds(c * _CHUNK, 16)]
        ucol = uid & 127
        icol = iid & 127
        acc = jnp.zeros((16,), jnp.float32)
        for j in range(_DIM):
            uu = plsc.load_gather(ubuf, [row_j + j, ucol])
            vv = plsc.load_gather(ibuf, [row_j + j, icol])
            acc = acc + uu * vv
        plsc.store_compressed(out_v.at[pl.ds(c * _CHUNK, 16)], acc,
                              mask=lane < _CHUNK)

    fire_chunk(0, 0)

    def step(cc, _):
        fire_chunk(cc * 2 + 1, 1)
        drain_compute_chunk(cc * 2, 0)
        fire_chunk(cc * 2 + 2, 0)
        drain_compute_chunk(cc * 2 + 1, 1)
        return _

    lax.fori_loop(0, _NCHUNK // 2 - 1, step, None)

    fire_chunk(_NCHUNK - 1, 1)
    drain_compute_chunk(_NCHUNK - 2, 0)
    drain_compute_chunk(_NCHUNK - 1, 1)

    pltpu.sync_copy(out_v.at[pl.ds(0, _B_PER_W)],
                    out_hbm.at[pl.ds(base, _B_PER_W)])


@jax.jit
def _mf(user_ids, item_ids, user_emb, item_emb):
    mesh = plsc.VectorSubcoreMesh(core_axis_name="c", subcore_axis_name="s")
    return pl.kernel(
        _mf_body,
        out_type=jax.ShapeDtypeStruct((_BATCH,), jnp.float32),
        mesh=mesh,
        scratch_types=[
            pltpu.VMEM((_B_PER_W + 16,), jnp.int32),
            pltpu.VMEM((_B_PER_W + 16,), jnp.int32),
            pltpu.VMEM((_CHUNK * _DIM, 128), jnp.float32),
            pltpu.VMEM((_CHUNK * _DIM, 128), jnp.float32),
            pltpu.VMEM((_CHUNK * _DIM, 128), jnp.float32),
            pltpu.VMEM((_CHUNK * _DIM, 128), jnp.float32),
            pltpu.VMEM((_B_PER_W + 16,), jnp.float32),
            pltpu.SemaphoreType.DMA,
            pltpu.SemaphoreType.DMA,
            pltpu.SemaphoreType.DMA,
            pltpu.SemaphoreType.DMA,
        ],
        compiler_params=pltpu.CompilerParams(
            needs_layout_passes=False, disable_bounds_checks=True),
    )(user_ids, item_ids, user_emb.T, item_emb.T)


def kernel(user_ids, item_ids, user_emb, item_emb):
    return _mf(user_ids, item_ids, user_emb, item_emb)


# speedup vs baseline: 4.3414x; 1.1124x over previous
import jax
import jax.numpy as jnp
from jax import lax
from jax.experimental import pallas as pl
from jax.experimental.pallas import tpu as pltpu
from jax.experimental.pallas import tpu_sc as plsc

_BATCH = 16384
_DIM = 32
_NUM_WORKERS = 32
_B_PER_W = _BATCH // _NUM_WORKERS
_CHUNK = 4
_NCHUNK = _B_PER_W // _CHUNK


def _mf_body(user_ids_hbm, item_ids_hbm, uemb_t_hbm, iemb_t_hbm,
             out_hbm, uid_v, iid_v, ubuf0, ibuf0, ubuf1, ibuf1,
             ubuf2, ibuf2, out_v,
             sem_u0, sem_i0, sem_u1, sem_i1, sem_u2, sem_i2):
    num_cores = 2
    wid = lax.axis_index("s") * num_cores + lax.axis_index("c")
    base = wid * _B_PER_W

    pltpu.sync_copy(user_ids_hbm.at[pl.ds(base, _B_PER_W)],
                    uid_v.at[pl.ds(0, _B_PER_W)])
    pltpu.sync_copy(item_ids_hbm.at[pl.ds(base, _B_PER_W)],
                    iid_v.at[pl.ds(0, _B_PER_W)])
    lane = lax.iota(jnp.int32, 16)
    row_j = (lane & (_CHUNK - 1)) * _DIM
    bufs = ((ubuf0, ibuf0, sem_u0, sem_i0), (ubuf1, ibuf1, sem_u1, sem_i1),
            (ubuf2, ibuf2, sem_u2, sem_i2))

    def fire_chunk(c, slot):
        ubuf, ibuf, sem_u, sem_i = bufs[slot]
        uid = uid_v[pl.ds(c * _CHUNK, 16)]
        iid = iid_v[pl.ds(c * _CHUNK, 16)]
        ubv = (uid >> 7) << 7
        ibv = (iid >> 7) << 7
        for b in range(_CHUNK):
            ub = jnp.sum(jnp.where(lane == b, ubv, 0))
            ib = jnp.sum(jnp.where(lane == b, ibv, 0))
            pltpu.async_copy(
                uemb_t_hbm.at[:, pl.ds(pl.multiple_of(ub, 128), 128)],
                ubuf.at[pl.ds(b * _DIM, _DIM), :], sem_u)
            pltpu.async_copy(
                iemb_t_hbm.at[:, pl.ds(pl.multiple_of(ib, 128), 128)],
                ibuf.at[pl.ds(b * _DIM, _DIM), :], sem_i)

    def drain_compute_chunk(c, slot):
        ubuf, ibuf, sem_u, sem_i = bufs[slot]
        for b in range(_CHUNK):
            pltpu.make_async_copy(
                uemb_t_hbm.at[:, pl.ds(0, 128)],
                ubuf.at[pl.ds(b * _DIM, _DIM), :], sem_u).wait()
            pltpu.make_async_copy(
                iemb_t_hbm.at[:, pl.ds(0, 128)],
                ibuf.at[pl.ds(b * _DIM, _DIM), :], sem_i).wait()
        uid = uid_v[pl.ds(c * _CHUNK, 16)]
        iid = iid_v[pl.ds(c * _CHUNK, 16)]
        ucol = uid & 127
        icol = iid & 127
        acc = jnp.zeros((16,), jnp.float32)
        for j in range(_DIM):
            uu = plsc.load_gather(ubuf, [row_j + j, ucol])
            vv = plsc.load_gather(ibuf, [row_j + j, icol])
            acc = acc + uu * vv
        plsc.store_compressed(out_v.at[pl.ds(c * _CHUNK, 16)], acc,
                              mask=lane < _CHUNK)

    fire_chunk(0, 0)
    fire_chunk(1, 1)

    def step(k, _):
        c = k * 3
        fire_chunk(c + 2, 2)
        drain_compute_chunk(c, 0)
        fire_chunk(c + 3, 0)
        drain_compute_chunk(c + 1, 1)
        fire_chunk(c + 4, 1)
        drain_compute_chunk(c + 2, 2)
        return _

    lax.fori_loop(0, _NCHUNK // 3, step, None)

    drain_compute_chunk(_NCHUNK - 2, 0)
    drain_compute_chunk(_NCHUNK - 1, 1)

    pltpu.sync_copy(out_v.at[pl.ds(0, _B_PER_W)],
                    out_hbm.at[pl.ds(base, _B_PER_W)])


@jax.jit
def _mf(user_ids, item_ids, user_emb, item_emb):
    mesh = plsc.VectorSubcoreMesh(core_axis_name="c", subcore_axis_name="s")
    return pl.kernel(
        _mf_body,
        out_type=jax.ShapeDtypeStruct((_BATCH,), jnp.float32),
        mesh=mesh,
        scratch_types=[
            pltpu.VMEM((_B_PER_W + 16,), jnp.int32),
            pltpu.VMEM((_B_PER_W + 16,), jnp.int32),
            pltpu.VMEM((_CHUNK * _DIM, 128), jnp.float32),
            pltpu.VMEM((_CHUNK * _DIM, 128), jnp.float32),
            pltpu.VMEM((_CHUNK * _DIM, 128), jnp.float32),
            pltpu.VMEM((_CHUNK * _DIM, 128), jnp.float32),
            pltpu.VMEM((_CHUNK * _DIM, 128), jnp.float32),
            pltpu.VMEM((_CHUNK * _DIM, 128), jnp.float32),
            pltpu.VMEM((_B_PER_W + 16,), jnp.float32),
            pltpu.SemaphoreType.DMA,
            pltpu.SemaphoreType.DMA,
            pltpu.SemaphoreType.DMA,
            pltpu.SemaphoreType.DMA,
            pltpu.SemaphoreType.DMA,
            pltpu.SemaphoreType.DMA,
        ],
        compiler_params=pltpu.CompilerParams(
            needs_layout_passes=False, disable_bounds_checks=True),
    )(user_ids, item_ids, user_emb.T, item_emb.T)


def kernel(user_ids, item_ids, user_emb, item_emb):
    return _mf(user_ids, item_ids, user_emb, item_emb)
